# baseline (speedup 1.0000x reference)
import functools

import jax
import jax.numpy as jnp
from jax import lax
from jax.experimental import pallas as pl
from jax.experimental.pallas import tpu as pltpu
from jax.experimental.pallas import tpu_sc as plsc

_NC = 2
_NS = 16
_NBUF = 7
_RA = 2


def kernel(hidden_states, position_embeddings):
    B, S, D = hidden_states.shape
    assert position_embeddings.shape == (S, D)
    NW = _NC * _NS
    rows_per_w = S // NW
    CH = 16
    n_ch = rows_per_w // CH
    mesh = plsc.VectorSubcoreMesh(core_axis_name="c", subcore_axis_name="s")

    @functools.partial(
        pl.kernel,
        mesh=mesh,
        out_type=jax.ShapeDtypeStruct((B, S, D), jnp.float32),
        scratch_types=[
            *([pltpu.VMEM((CH, D), jnp.float32)] * _NBUF),
            pltpu.SemaphoreType.DMA,
            *([pltpu.SemaphoreType.DMA] * _NBUF),
        ],
    )
    def sc_bcast(table_hbm, out_hbm, *rest):
        bufs = rest[:_NBUF]
        rsem = rest[_NBUF]
        wsems = rest[_NBUF + 1:]
        wid = lax.axis_index("s") * _NC + lax.axis_index("c")
        base = wid * rows_per_w

        def rd(i):
            return pltpu.make_async_copy(
                table_hbm.at[pl.ds(base + i * CH, CH)], bufs[i % _NBUF], rsem
            )

        def wr(i, b):
            return pltpu.make_async_copy(
                bufs[i % _NBUF],
                out_hbm.at[b, pl.ds(base + i * CH, CH)],
                wsems[i % _NBUF],
            )

        for i in range(min(_RA, n_ch)):
            rd(i).start()
        for i in range(n_ch):
            rd(i).wait()
            if i + _RA < n_ch:
                j = i + _RA - _NBUF
                if j >= 0:
                    for b in range(B):
                        wr(j, b).wait()
                rd(i + _RA).start()
            for b in range(B):
                wr(i, b).start()
        for i in range(max(0, n_ch - _NBUF), n_ch):
            for b in range(B):
                wr(i, b).wait()

    return sc_bcast(position_embeddings)


# speedup vs baseline: 1.0636x; 1.0636x over previous
import functools

import jax
import jax.numpy as jnp
from jax import lax
from jax.experimental import pallas as pl
from jax.experimental.pallas import tpu as pltpu
from jax.experimental.pallas import tpu_sc as plsc

_NC = 2
_NS = 16
_NBUF = 3
_RA = 2


def kernel(hidden_states, position_embeddings):
    B, S, D = hidden_states.shape
    assert position_embeddings.shape == (S, D)
    NW = _NC * _NS
    rows_per_w = S // NW
    CH = 32
    n_ch = rows_per_w // CH
    mesh = plsc.VectorSubcoreMesh(core_axis_name="c", subcore_axis_name="s")

    @functools.partial(
        pl.kernel,
        mesh=mesh,
        out_type=jax.ShapeDtypeStruct((B, S, D), jnp.float32),
        scratch_types=[
            *([pltpu.VMEM((CH, D), jnp.float32)] * _NBUF),
            pltpu.SemaphoreType.DMA,
            *([pltpu.SemaphoreType.DMA] * _NBUF),
        ],
    )
    def sc_bcast(table_hbm, out_hbm, *rest):
        bufs = rest[:_NBUF]
        rsem = rest[_NBUF]
        wsems = rest[_NBUF + 1:]
        wid = lax.axis_index("s") * _NC + lax.axis_index("c")
        base = wid * rows_per_w

        def rd(i):
            return pltpu.make_async_copy(
                table_hbm.at[pl.ds(base + i * CH, CH)], bufs[i % _NBUF], rsem
            )

        def wr(i, b):
            return pltpu.make_async_copy(
                bufs[i % _NBUF],
                out_hbm.at[b, pl.ds(base + i * CH, CH)],
                wsems[i % _NBUF],
            )

        for i in range(min(_RA, n_ch)):
            rd(i).start()
        for i in range(n_ch):
            rd(i).wait()
            if i + _RA < n_ch:
                j = i + _RA - _NBUF
                if j >= 0:
                    for b in range(B):
                        wr(j, b).wait()
                rd(i + _RA).start()
            for b in range(B):
                wr(i, b).start()
        for i in range(max(0, n_ch - _NBUF), n_ch):
            for b in range(B):
                wr(i, b).wait()

    return sc_bcast(position_embeddings)
